# baseline (speedup 1.0000x reference)
import functools

import jax
import jax.numpy as jnp
from jax import lax
from jax.experimental import pallas as pl
from jax.experimental.pallas import tpu as pltpu
from jax.experimental.pallas import tpu_sc as plsc

B = 1024
L1 = 26
L2 = 20
D = 32
NBLK = L1 * L2
NC = 2
NS = 16
NW = NC * NS
ROUNDS = NBLK // NW
TAIL = NBLK - ROUNDS * NW

_mesh = plsc.VectorSubcoreMesh(core_axis_name="c", subcore_axis_name="s")


@functools.partial(
    pl.kernel,
    mesh=_mesh,
    out_type=jax.ShapeDtypeStruct((NBLK, 4, 8, 8, 128), jnp.float32),
    scratch_types=[
        pltpu.VMEM((2, B), jnp.int32),
        pltpu.VMEM((2, B), jnp.int32),
        pltpu.VMEM((B, D), jnp.float32),
        pltpu.VMEM((B, D), jnp.float32),
        pltpu.VMEM((4, 8, 8, 128), jnp.float32),
        pltpu.SemaphoreType.DMA,
        pltpu.SemaphoreType.DMA,
    ],
    compiler_params=pltpu.CompilerParams(
        use_tc_tiling_on_sc=False, needs_layout_passes=False
    ),
)
def _gather_kernel(
    table_hbm, xf_hbm, out_hbm, iidx_v, idx_v, rows_v0, rows_v1, t_v, gsem, isem
):
    w = lax.axis_index("s") * NC + lax.axis_index("c")
    iota16 = lax.iota(jnp.int32, 16)
    vstep = iota16 * NBLK
    rows_bufs = (rows_v0, rows_v1)

    def fire_idx(slot, k):
        def body(i, carry):
            iidx_v[slot, pl.ds(i * 16, 16)] = vstep + (i * 16 * NBLK + k)
            return carry

        lax.fori_loop(0, 64, body, 0)
        return pltpu.async_copy(xf_hbm.at[iidx_v.at[slot]], idx_v.at[slot], isem)

    def fire_rows(slot):
        return pltpu.async_copy(table_hbm.at[idx_v.at[slot]], rows_bufs[slot], gsem)

    def transpose(slot):
        rows = rows_bufs[slot]

        def body(m, carry):
            rowvec = m * 16 + iota16
            blvec = (m % 8) * 16 + iota16
            bhvec = jnp.full((16,), 0, jnp.int32) + m // 8
            for d0 in range(D):
                dvec = (iota16 + d0) % D
                vals = plsc.load_gather(rows, [rowvec, dvec])
                plsc.store_scatter(t_v, [dvec // 8, bhvec, dvec % 8, blvec], vals)
            return carry

        lax.fori_loop(0, 64, body, 0)

    def blk(r):
        return w + NW * r

    idxg = [fire_idx(0, blk(0)), None]
    idxg[1] = fire_idx(1, blk(1))
    idxg[0].wait()
    rowsg = [fire_rows(0), None]

    for r in range(ROUNDS):
        slot = r % 2
        nslot = (r + 1) % 2
        if r + 1 < ROUNDS:
            idxg[nslot].wait()
            rowsg[nslot] = fire_rows(nslot)
        rowsg[slot].wait()
        if r + 2 < ROUNDS:
            idxg[slot] = fire_idx(slot, blk(r + 2))
        transpose(slot)
        pltpu.sync_copy(t_v, out_hbm.at[blk(r)])

    @pl.when(w < TAIL)
    def _():
        k = ROUNDS * NW + w
        fire_idx(0, k).wait()
        pltpu.async_copy(table_hbm.at[idx_v.at[0]], rows_v0, gsem).wait()
        transpose(0)
        pltpu.sync_copy(t_v, out_hbm.at[k])


def kernel(x, W):
    xf = x.reshape(-1).astype(jnp.int32)
    pal = _gather_kernel(W, xf)
    pal6 = pal.reshape(L1, L2, 4, 8, 8, 128)
    out = jnp.transpose(pal6, (3, 5, 0, 1, 2, 4)).reshape(B, L1, L2, D)
    return out
